# Initial kernel scaffold; baseline (speedup 1.0000x reference)
import jax
import jax.numpy as jnp
from jax.experimental import pallas as pl


def kernel(x, edge_index, W0, b0, W1, b1):
    raise NotImplementedError("write your pallas kernel here")


# speedup vs baseline: 9.7274x; 9.7274x over previous
import functools

import jax
import jax.numpy as jnp
from jax import lax
from jax.experimental import pallas as pl
from jax.experimental.pallas import tpu as pltpu
from jax.experimental.pallas import tpu_sc as plsc

N = 10000
NP = 10240
E = 320000
DI = 128
DH = 256
DC = 40
DCP = 128
DW = 16

NC = 2
NS = 16
NW = NC * NS
G = 128
NGP = 2560
GPT = NGP // NW
NBUF = 2
CH = 16
NSUP = CH // NBUF
NCHK = GPT // CH
ROWS_PER_TILE = NP // NS

_mesh = plsc.VectorSubcoreMesh(
    core_axis_name="c", subcore_axis_name="s", num_cores=NC, num_subcores=NS)


def _sc_degrees(edge3, e0, e64, zeros128):
    @functools.partial(
        pl.kernel,
        out_type=jax.ShapeDtypeStruct((NC, NP, DI), jnp.float32),
        mesh=_mesh,
        scratch_types=[
            pltpu.VMEM((2, CH, G), jnp.int32),
            pltpu.VMEM((G, DI), jnp.float32),
            pltpu.VMEM((G, DI), jnp.float32),
            pltpu.VMEM_SHARED((NP, DI), jnp.float32),
            pltpu.SemaphoreType.DMA,
        ],
    )
    def k(edge_hbm, e0_hbm, e64_hbm, z_hbm, out_hbm, idxb, ones0, ones64, acc,
          ssem):
        c = lax.axis_index("c")
        s = lax.axis_index("s")
        flat = s * NC + c
        r0 = s * ROWS_PER_TILE
        pltpu.sync_copy(z_hbm, acc.at[pl.ds(r0, ROWS_PER_TILE)])
        pltpu.sync_copy(e0_hbm, ones0)
        pltpu.sync_copy(e64_hbm, ones64)
        g0 = flat * GPT
        plsc.subcore_barrier()

        def chunk(ci, _):
            cg0 = g0 + ci * CH
            pltpu.sync_copy(edge_hbm.at[0, pl.ds(cg0, CH)], idxb.at[0])
            pltpu.sync_copy(edge_hbm.at[1, pl.ds(cg0, CH)], idxb.at[1])

            def sup(i, _):
                base = i * NBUF
                sds = []
                for j in range(NBUF):
                    sds.append(pltpu.async_copy(
                        ones0, acc.at[idxb.at[0, base + j]], ssem, add=True))
                    sds.append(pltpu.async_copy(
                        ones64, acc.at[idxb.at[1, base + j]], ssem, add=True))
                for d in sds:
                    d.wait()
                return 0

            lax.fori_loop(0, NSUP, sup, 0)
            return 0

        lax.fori_loop(0, NCHK, chunk, 0)
        plsc.subcore_barrier()
        pltpu.sync_copy(acc.at[pl.ds(r0, ROWS_PER_TILE)],
                        out_hbm.at[c, pl.ds(r0, ROWS_PER_TILE)])

    return k(edge3, e0, e64, zeros128)


def _sc_propagate(src, edge3, zeros, src_sel, dst_sel, width):
    @functools.partial(
        pl.kernel,
        out_type=jax.ShapeDtypeStruct((NC, NP, width), jnp.float32),
        mesh=_mesh,
        scratch_types=[
            pltpu.VMEM((2, CH, G), jnp.int32),
            pltpu.VMEM((NBUF, G, width), jnp.float32),
            pltpu.VMEM_SHARED((NP, width), jnp.float32),
            pltpu.SemaphoreType.DMA,
            pltpu.SemaphoreType.DMA,
        ],
    )
    def k(src_hbm, edge_hbm, z_hbm, out_hbm, idxb, rows, acc, gsem, ssem):
        c = lax.axis_index("c")
        s = lax.axis_index("s")
        flat = s * NC + c
        r0 = s * ROWS_PER_TILE
        pltpu.sync_copy(z_hbm, acc.at[pl.ds(r0, ROWS_PER_TILE)])
        g0 = flat * GPT
        plsc.subcore_barrier()

        def chunk(ci, _):
            cg0 = g0 + ci * CH
            pltpu.sync_copy(edge_hbm.at[src_sel, pl.ds(cg0, CH)], idxb.at[0])
            pltpu.sync_copy(edge_hbm.at[dst_sel, pl.ds(cg0, CH)], idxb.at[1])

            def sup(i, _):
                base = i * NBUF
                gds = []
                for j in range(NBUF):
                    gds.append(pltpu.async_copy(
                        src_hbm.at[idxb.at[0, base + j]], rows.at[j], gsem))
                sds = []
                for j in range(NBUF):
                    gds[j].wait()
                    sds.append(pltpu.async_copy(
                        rows.at[j], acc.at[idxb.at[1, base + j]], ssem,
                        add=True))
                for d in sds:
                    d.wait()
                return 0

            lax.fori_loop(0, NSUP, sup, 0)
            return 0

        lax.fori_loop(0, NCHK, chunk, 0)
        plsc.subcore_barrier()
        pltpu.sync_copy(acc.at[pl.ds(r0, ROWS_PER_TILE)],
                        out_hbm.at[c, pl.ds(r0, ROWS_PER_TILE)])

    return k(src, edge3, zeros)


_BLK = 2048


def _tc_scale(dpb, x):
    def body(dpb_ref, x_ref, dis_ref, binv_ref, xs_ref):
        d = dpb_ref[0][:, 0:1] + dpb_ref[1][:, 0:1]
        b = dpb_ref[0][:, 64:65] + dpb_ref[1][:, 64:65]
        dis = jnp.where(d > 0, lax.rsqrt(jnp.maximum(d, 1e-30)), 0.0)
        binv = jnp.where(b > 0, 1.0 / jnp.maximum(b, 1e-30), 0.0)
        dis_ref[...] = dis
        binv_ref[...] = binv
        xs_ref[...] = x_ref[...] * dis

    return pl.pallas_call(
        body,
        grid=(NP // _BLK,),
        in_specs=[
            pl.BlockSpec((NC, _BLK, DI), lambda i: (0, i, 0)),
            pl.BlockSpec((_BLK, DI), lambda i: (i, 0)),
        ],
        out_specs=[
            pl.BlockSpec((_BLK, 1), lambda i: (i, 0)),
            pl.BlockSpec((_BLK, 1), lambda i: (i, 0)),
            pl.BlockSpec((_BLK, DI), lambda i: (i, 0)),
        ],
        out_shape=[
            jax.ShapeDtypeStruct((NP, 1), jnp.float32),
            jax.ShapeDtypeStruct((NP, 1), jnp.float32),
            jax.ShapeDtypeStruct((NP, DI), jnp.float32),
        ],
    )(dpb, x)


def _tc_combine_scale(parts, scale, width):
    def body(p_ref, s_ref, o_ref):
        o_ref[...] = (p_ref[0] + p_ref[1]) * s_ref[...]

    return pl.pallas_call(
        body,
        grid=(NP // _BLK,),
        in_specs=[
            pl.BlockSpec((NC, _BLK, width), lambda i: (0, i, 0)),
            pl.BlockSpec((_BLK, 1), lambda i: (i, 0)),
        ],
        out_specs=pl.BlockSpec((_BLK, width), lambda i: (i, 0)),
        out_shape=jax.ShapeDtypeStruct((NP, width), jnp.float32),
    )(parts, scale)


def _tc_layer(q, dis, W0, b0, W1p):
    def body(q_ref, dis_ref, w0_ref, b0_ref, w1_ref, o_ref):
        px = (q_ref[0] + q_ref[1]) * dis_ref[...]
        h = jnp.dot(px, w0_ref[...], preferred_element_type=jnp.float32)
        h = h + b0_ref[...]
        h = jnp.where(h > 0, h, jnp.exp(jnp.minimum(h, 0.0)) - 1.0)
        o = jnp.dot(h, w1_ref[...], preferred_element_type=jnp.float32)
        o_ref[...] = o * dis_ref[...]

    return pl.pallas_call(
        body,
        grid=(NP // _BLK,),
        in_specs=[
            pl.BlockSpec((NC, _BLK, DI), lambda i: (0, i, 0)),
            pl.BlockSpec((_BLK, 1), lambda i: (i, 0)),
            pl.BlockSpec((DI, DH), lambda i: (0, 0)),
            pl.BlockSpec((1, DH), lambda i: (0, 0)),
            pl.BlockSpec((DH, DCP), lambda i: (0, 0)),
        ],
        out_specs=pl.BlockSpec((_BLK, DCP), lambda i: (i, 0)),
        out_shape=jax.ShapeDtypeStruct((NP, DCP), jnp.float32),
    )(q, dis, W0, b0, W1p)


def _tc_final(parts, dis, b1p):
    def body(p_ref, s_ref, b_ref, o_ref):
        o_ref[...] = (p_ref[0] + p_ref[1]) * s_ref[...] + b_ref[...]

    return pl.pallas_call(
        body,
        grid=(NP // _BLK,),
        in_specs=[
            pl.BlockSpec((NC, _BLK, DCP), lambda i: (0, i, 0)),
            pl.BlockSpec((_BLK, 1), lambda i: (i, 0)),
            pl.BlockSpec((1, DCP), lambda i: (0, 0)),
        ],
        out_specs=pl.BlockSpec((_BLK, DCP), lambda i: (i, 0)),
        out_shape=jax.ShapeDtypeStruct((NP, DCP), jnp.float32),
    )(parts, dis, b1p)


def kernel(x, edge_index, W0, b0, W1, b1):
    edge_index = edge_index.astype(jnp.int32)

    npad = NGP * G - E
    pad_idx = (N + (jnp.arange(npad, dtype=jnp.int32) % (NP - N)))
    pad2 = jnp.stack([pad_idx, pad_idx])
    edge3 = jnp.concatenate([edge_index, pad2], axis=1).reshape(2, NGP, G)

    e0 = jnp.zeros((G, DI), jnp.float32).at[:, 0].set(1.0)
    e64 = jnp.zeros((G, DI), jnp.float32).at[:, 64].set(1.0)
    z128 = jnp.zeros((ROWS_PER_TILE, DI), jnp.float32)
    zc = jnp.zeros((ROWS_PER_TILE, DCP), jnp.float32)
    W1p = jnp.pad(W1, ((0, 0), (0, DCP - DC)))
    b0r = b0.reshape(1, DH)
    b1p = jnp.pad(b1, (0, DCP - DC)).reshape(1, DCP)
    xp = jnp.pad(x, ((0, NP - N), (0, 0)))

    dpb = _sc_degrees(edge3, e0, e64, z128)
    dis, binv, xs = _tc_scale(dpb, xp)

    p = _sc_propagate(xs, edge3, z128, 0, 1, DI)
    m = _tc_combine_scale(p, binv, DI)
    q = _sc_propagate(m, edge3, z128, 1, 0, DI)
    hs2 = _tc_layer(q, dis, W0, b0r, W1p)

    p2 = _sc_propagate(hs2, edge3, zc, 0, 1, DCP)
    m2 = _tc_combine_scale(p2, binv, DCP)
    q2 = _sc_propagate(m2, edge3, zc, 1, 0, DCP)
    out = _tc_final(q2, dis, b1p)
    return out[:N, :DC]
